# baseline (speedup 1.0000x reference)
import functools
import jax
import jax.numpy as jnp
from jax import lax
from jax.experimental import pallas as pl
from jax.experimental.pallas import tpu as pltpu, tpu_sc as plsc

NUM_EXPERTS = 64
TOP_K = 8
BLOCK_M = 1024
LANES = 16


def _scores_block(x_ref, w_ref, p_ref):
    logits = lax.dot_general(
        x_ref[...], w_ref[...],
        dimension_numbers=(((1,), (0,)), ((), ())),
        preferred_element_type=jnp.float32,
    )
    m = jnp.max(logits, axis=1, keepdims=True)
    e = jnp.exp(logits - m)
    p_ref[...] = e / jnp.sum(e, axis=1, keepdims=True)


def _tc_scores(h, Wt):
    M, K = h.shape
    E = Wt.shape[1]
    bm = BLOCK_M
    return pl.pallas_call(
        _scores_block,
        grid=(M // bm,),
        in_specs=[
            pl.BlockSpec((bm, K), lambda i: (i, 0)),
            pl.BlockSpec((K, E), lambda i: (0, 0)),
        ],
        out_specs=pl.BlockSpec((bm, E), lambda i: (i, 0)),
        out_shape=jax.ShapeDtypeStruct((M, E), jnp.float32),
    )(h, Wt)


def _make_sc_topk(M):
    info = plsc.get_sparse_core_info()
    NC, NS = info.num_cores, info.num_subcores
    NW = NC * NS
    tw = M // NW
    ngroups = tw // LANES
    mesh = plsc.VectorSubcoreMesh(core_axis_name="c", subcore_axis_name="s")

    @functools.partial(
        pl.kernel,
        mesh=mesh,
        out_type=[
            jax.ShapeDtypeStruct((TOP_K, M), jnp.float32),
            jax.ShapeDtypeStruct((TOP_K, M), jnp.int32),
        ],
        scratch_types=[
            pltpu.VMEM((NUM_EXPERTS, tw), jnp.float32),
            pltpu.VMEM((TOP_K, tw), jnp.float32),
            pltpu.VMEM((TOP_K, tw), jnp.int32),
        ],
    )
    def sc_topk(pT_hbm, wout_hbm, iout_hbm, slab, wv, iv):
        wid = lax.axis_index("s") * NC + lax.axis_index("c")
        base = wid * tw
        pltpu.sync_copy(pT_hbm.at[:, pl.ds(base, tw)], slab)

        def group_body(g, _):
            neg = jnp.full((LANES,), -1.0, jnp.float32)
            zero = jnp.zeros((LANES,), jnp.int32)
            tv = [neg] * TOP_K
            ti = [zero] * TOP_K
            for e in range(NUM_EXPERTS):
                cols = jnp.full((LANES,), e, jnp.int32)
                cv = slab[e, pl.ds(g * LANES, LANES)]
                ci = cols
                for j in range(TOP_K):
                    swap = cv > tv[j]
                    nv = jnp.where(swap, cv, tv[j])
                    cv = jnp.where(swap, tv[j], cv)
                    tv[j] = nv
                    ni = jnp.where(swap, ci, ti[j])
                    ci = jnp.where(swap, ti[j], ci)
                    ti[j] = ni
            for j in range(TOP_K):
                wv[j, pl.ds(g * LANES, LANES)] = tv[j]
                iv[j, pl.ds(g * LANES, LANES)] = ti[j]
            return 0

        lax.fori_loop(0, ngroups, group_body, 0)
        pltpu.sync_copy(wv, wout_hbm.at[:, pl.ds(base, tw)])
        pltpu.sync_copy(iv, iout_hbm.at[:, pl.ds(base, tw)])

    return sc_topk


def kernel(x, W):
    h = x.reshape(-1, x.shape[-1])
    M, K = h.shape
    Wt = jnp.swapaxes(W, 0, 1)
    p = _tc_scores(h, Wt)
    pT = jnp.swapaxes(p, 0, 1)
    wT, iT = _make_sc_topk(M)(pT)
    return (jnp.swapaxes(wT, 0, 1), jnp.swapaxes(iT, 0, 1))


# speedup vs baseline: 1.0182x; 1.0182x over previous
import functools
import jax
import jax.numpy as jnp
from jax import lax
from jax.experimental import pallas as pl
from jax.experimental.pallas import tpu as pltpu, tpu_sc as plsc

NUM_EXPERTS = 64
TOP_K = 8
BLOCK_M = 1024
LANES = 16


def _scores_block(x_ref, w_ref, pT_ref):
    logits = lax.dot_general(
        x_ref[...], w_ref[...],
        dimension_numbers=(((1,), (0,)), ((), ())),
        preferred_element_type=jnp.float32,
    )
    m = jnp.max(logits, axis=1, keepdims=True)
    e = jnp.exp(logits - m)
    p = e / jnp.sum(e, axis=1, keepdims=True)
    pT_ref[...] = jnp.swapaxes(p, 0, 1)


def _tc_scores(h, Wt):
    M, K = h.shape
    E = Wt.shape[1]
    bm = BLOCK_M
    return pl.pallas_call(
        _scores_block,
        grid=(M // bm,),
        in_specs=[
            pl.BlockSpec((bm, K), lambda i: (i, 0)),
            pl.BlockSpec((K, E), lambda i: (0, 0)),
        ],
        out_specs=pl.BlockSpec((E, bm), lambda i: (0, i)),
        out_shape=jax.ShapeDtypeStruct((E, M), jnp.float32),
    )(h, Wt)


def _make_sc_topk(M):
    info = plsc.get_sparse_core_info()
    NC, NS = info.num_cores, info.num_subcores
    NW = NC * NS
    tw = M // NW
    ngroups = tw // LANES
    mesh = plsc.VectorSubcoreMesh(core_axis_name="c", subcore_axis_name="s")

    @functools.partial(
        pl.kernel,
        mesh=mesh,
        out_type=[
            jax.ShapeDtypeStruct((TOP_K, M), jnp.float32),
            jax.ShapeDtypeStruct((TOP_K, M), jnp.int32),
        ],
        scratch_types=[
            pltpu.VMEM((NUM_EXPERTS, tw), jnp.float32),
            pltpu.VMEM((TOP_K, tw), jnp.float32),
            pltpu.VMEM((TOP_K, tw), jnp.int32),
        ],
    )
    def sc_topk(pT_hbm, wout_hbm, iout_hbm, slab, wv, iv):
        wid = lax.axis_index("s") * NC + lax.axis_index("c")
        base = wid * tw
        pltpu.sync_copy(pT_hbm.at[:, pl.ds(base, tw)], slab)

        def group_body(g, _):
            neg = jnp.full((LANES,), -1.0, jnp.float32)
            zero = jnp.zeros((LANES,), jnp.int32)
            tv = [neg] * TOP_K
            ti = [zero] * TOP_K
            for e in range(NUM_EXPERTS):
                cols = jnp.full((LANES,), e, jnp.int32)
                cv = slab[e, pl.ds(g * LANES, LANES)]
                ci = cols
                for j in range(TOP_K):
                    swap = cv > tv[j]
                    nv = jnp.where(swap, cv, tv[j])
                    cv = jnp.where(swap, tv[j], cv)
                    tv[j] = nv
                    ni = jnp.where(swap, ci, ti[j])
                    ci = jnp.where(swap, ti[j], ci)
                    ti[j] = ni
            for j in range(TOP_K):
                wv[j, pl.ds(g * LANES, LANES)] = tv[j]
                iv[j, pl.ds(g * LANES, LANES)] = ti[j]
            return 0

        lax.fori_loop(0, ngroups, group_body, 0)
        pltpu.sync_copy(wv, wout_hbm.at[:, pl.ds(base, tw)])
        pltpu.sync_copy(iv, iout_hbm.at[:, pl.ds(base, tw)])

    return sc_topk


def kernel(x, W):
    h = x.reshape(-1, x.shape[-1])
    M, K = h.shape
    Wt = jnp.swapaxes(W, 0, 1)
    pT = _tc_scores(h, Wt)
    wT, iT = _make_sc_topk(M)(pT)
    return (jnp.swapaxes(wT, 0, 1), jnp.swapaxes(iT, 0, 1))
